# baseline (speedup 1.0000x reference)
import jax
import jax.numpy as jnp
from jax.experimental import pallas as pl
from jax.experimental.pallas import tpu as pltpu

_NFEAT, _NHID, _NMETA, _DIM_MP, _NCLASS = 128, 64, 3, 32, 8
_ALPHA = 0.2
_N = 4096
_BM = 1024
_NB = _N // _BM
_HALF = _N // 2


def _hingcn_body(x_ref, adj0_ref, adj1_ref, w1_ref, b1_ref, w2_ref, b2_ref,
                 a_ref, wlin_ref, blin_ref, out_ref,
                 y1_s, h1_s, y2_s, e0_s, e1_s):
    m = pl.program_id(0)
    layer = pl.program_id(1)
    i = pl.program_id(2)
    row0 = i * _BM
    adj0 = adj0_ref[0]
    adj1 = adj1_ref[0]

    @pl.when(layer == 0)
    def _layer1():
        @pl.when(i == 0)
        def _compute_y1():
            y1_s[...] = jnp.dot(x_ref[...], w1_ref[m],
                                preferred_element_type=jnp.float32
                                ).astype(jnp.bfloat16)

        h = jnp.dot(adj0.astype(jnp.bfloat16), y1_s[pl.ds(0, _HALF), :],
                    preferred_element_type=jnp.float32)
        h += jnp.dot(adj1.astype(jnp.bfloat16), y1_s[pl.ds(_HALF, _HALF), :],
                     preferred_element_type=jnp.float32)
        h1_s[pl.ds(row0, _BM), :] = jnp.maximum(h + b1_ref[m],
                                                0.0).astype(jnp.bfloat16)

    @pl.when(layer == 1)
    def _layer2():
        @pl.when(i == 0)
        def _compute_y2():
            y2_s[...] = jnp.dot(h1_s[...], w2_ref[m],
                                preferred_element_type=jnp.float32
                                ).astype(jnp.bfloat16)

        e = jnp.dot(adj0.astype(jnp.bfloat16), y2_s[pl.ds(0, _HALF), :],
                    preferred_element_type=jnp.float32)
        e += jnp.dot(adj1.astype(jnp.bfloat16), y2_s[pl.ds(_HALF, _HALF), :],
                     preferred_element_type=jnp.float32)
        e = jnp.maximum(e + b2_ref[m], 0.0)

        @pl.when(m == 0)
        def _store0():
            e0_s[pl.ds(row0, _BM), :] = e

        @pl.when(m == 1)
        def _store1():
            e1_s[pl.ds(row0, _BM), :] = e

        @pl.when(m == _NMETA - 1)
        def _attend():
            e0 = e0_s[pl.ds(row0, _BM), :]
            e1 = e1_s[pl.ds(row0, _BM), :]
            a_v = a_ref[...]
            s0 = jnp.dot(e0, a_v, preferred_element_type=jnp.float32)
            s1 = jnp.dot(e1, a_v, preferred_element_type=jnp.float32)
            s2 = jnp.dot(e, a_v, preferred_element_type=jnp.float32)
            s0 = jnp.where(s0 > 0, s0, _ALPHA * s0)
            s1 = jnp.where(s1 > 0, s1, _ALPHA * s1)
            s2 = jnp.where(s2 > 0, s2, _ALPHA * s2)
            mx = jnp.maximum(s0, jnp.maximum(s1, s2))
            x0 = jnp.exp(s0 - mx)
            x1 = jnp.exp(s1 - mx)
            x2 = jnp.exp(s2 - mx)
            comb = (x0 * e0 + x1 * e1 + x2 * e) / (x0 + x1 + x2)
            logits = jnp.dot(comb, wlin_ref[...],
                             preferred_element_type=jnp.float32)
            logits = jnp.maximum(logits + blin_ref[...], 0.0)
            zmax = jnp.max(logits, axis=1, keepdims=True)
            z = logits - zmax
            out_ref[...] = z - jnp.log(jnp.sum(jnp.exp(z), axis=1,
                                               keepdims=True))


def kernel(input, adjs, W1, b1, W2, b2, a, Wlin, blin):
    b1r = b1.reshape(_NMETA, 1, _NHID)
    b2r = b2.reshape(_NMETA, 1, _DIM_MP)
    a_r = a.reshape(_DIM_MP, 1)
    blin_r = blin.reshape(1, _NCLASS)
    grid = (_NMETA, 2, _NB)
    return pl.pallas_call(
        _hingcn_body,
        grid=grid,
        in_specs=[
            pl.BlockSpec((_N, _NFEAT), lambda m, l, i: (0, 0)),
            pl.BlockSpec((1, _BM, _HALF), lambda m, l, i: (m, i, 0)),
            pl.BlockSpec((1, _BM, _HALF), lambda m, l, i: (m, i, 1)),
            pl.BlockSpec((_NMETA, _NFEAT, _NHID), lambda m, l, i: (0, 0, 0)),
            pl.BlockSpec((_NMETA, 1, _NHID), lambda m, l, i: (0, 0, 0)),
            pl.BlockSpec((_NMETA, _NHID, _DIM_MP), lambda m, l, i: (0, 0, 0)),
            pl.BlockSpec((_NMETA, 1, _DIM_MP), lambda m, l, i: (0, 0, 0)),
            pl.BlockSpec((_DIM_MP, 1), lambda m, l, i: (0, 0)),
            pl.BlockSpec((_DIM_MP, _NCLASS), lambda m, l, i: (0, 0)),
            pl.BlockSpec((1, _NCLASS), lambda m, l, i: (0, 0)),
        ],
        out_specs=pl.BlockSpec((_BM, _NCLASS), lambda m, l, i: (i, 0)),
        out_shape=jax.ShapeDtypeStruct((_N, _NCLASS), jnp.float32),
        compiler_params=pltpu.CompilerParams(
            vmem_limit_bytes=62 * 1024 * 1024,
        ),
        scratch_shapes=[
            pltpu.VMEM((_N, _NHID), jnp.bfloat16),
            pltpu.VMEM((_N, _NHID), jnp.bfloat16),
            pltpu.VMEM((_N, _DIM_MP), jnp.bfloat16),
            pltpu.VMEM((_N, _DIM_MP), jnp.float32),
            pltpu.VMEM((_N, _DIM_MP), jnp.float32),
        ],
    )(input, adjs, adjs, W1, b1r, W2, b2r, a_r, Wlin, blin_r)


# speedup vs baseline: 1.0294x; 1.0294x over previous
import jax
import jax.numpy as jnp
from jax.experimental import pallas as pl
from jax.experimental.pallas import tpu as pltpu

_NFEAT, _NHID, _NMETA, _DIM_MP, _NCLASS = 128, 64, 3, 32, 8
_ALPHA = 0.2
_N = 4096
_BM = 1024
_NB = _N // _BM
_HALF = _N // 2


def _hingcn_body(x_ref, adj_ref, w1_ref, b1_ref, w2_ref, b2_ref,
                 a_ref, wlin_ref, blin_ref, out_ref,
                 y1_s, h1_s, y2_s, e0_s, e1_s):
    m = pl.program_id(0)
    layer = pl.program_id(1)
    i = pl.program_id(2)
    row0 = i * _BM
    adj = adj_ref[0]

    @pl.when(layer == 0)
    def _layer1():
        @pl.when(i == 0)
        def _compute_y1():
            y1_s[...] = jnp.dot(x_ref[...], w1_ref[m],
                                preferred_element_type=jnp.float32
                                ).astype(jnp.bfloat16)

        h = jnp.dot(adj.astype(jnp.bfloat16), y1_s[...],
                    preferred_element_type=jnp.float32)
        h1_s[pl.ds(row0, _BM), :] = jnp.maximum(h + b1_ref[m],
                                                0.0).astype(jnp.bfloat16)

    @pl.when(layer == 1)
    def _layer2():
        @pl.when(i == 0)
        def _compute_y2():
            y2_s[...] = jnp.dot(h1_s[...], w2_ref[m],
                                preferred_element_type=jnp.float32
                                ).astype(jnp.bfloat16)

        e = jnp.dot(adj.astype(jnp.bfloat16), y2_s[...],
                    preferred_element_type=jnp.float32)
        e = jnp.maximum(e + b2_ref[m], 0.0)

        @pl.when(m == 0)
        def _store0():
            e0_s[pl.ds(row0, _BM), :] = e

        @pl.when(m == 1)
        def _store1():
            e1_s[pl.ds(row0, _BM), :] = e

        @pl.when(m == _NMETA - 1)
        def _attend():
            e0 = e0_s[pl.ds(row0, _BM), :]
            e1 = e1_s[pl.ds(row0, _BM), :]
            a_v = a_ref[...]
            s0 = jnp.dot(e0, a_v, preferred_element_type=jnp.float32)
            s1 = jnp.dot(e1, a_v, preferred_element_type=jnp.float32)
            s2 = jnp.dot(e, a_v, preferred_element_type=jnp.float32)
            s0 = jnp.where(s0 > 0, s0, _ALPHA * s0)
            s1 = jnp.where(s1 > 0, s1, _ALPHA * s1)
            s2 = jnp.where(s2 > 0, s2, _ALPHA * s2)
            mx = jnp.maximum(s0, jnp.maximum(s1, s2))
            x0 = jnp.exp(s0 - mx)
            x1 = jnp.exp(s1 - mx)
            x2 = jnp.exp(s2 - mx)
            comb = (x0 * e0 + x1 * e1 + x2 * e) / (x0 + x1 + x2)
            logits = jnp.dot(comb, wlin_ref[...],
                             preferred_element_type=jnp.float32)
            logits = jnp.maximum(logits + blin_ref[...], 0.0)
            zmax = jnp.max(logits, axis=1, keepdims=True)
            z = logits - zmax
            out_ref[...] = z - jnp.log(jnp.sum(jnp.exp(z), axis=1,
                                               keepdims=True))


def kernel(input, adjs, W1, b1, W2, b2, a, Wlin, blin):
    b1r = b1.reshape(_NMETA, 1, _NHID)
    b2r = b2.reshape(_NMETA, 1, _DIM_MP)
    a_r = a.reshape(_DIM_MP, 1)
    blin_r = blin.reshape(1, _NCLASS)
    grid = (_NMETA, 2, _NB)
    return pl.pallas_call(
        _hingcn_body,
        grid=grid,
        in_specs=[
            pl.BlockSpec((_N, _NFEAT), lambda m, l, i: (0, 0)),
            pl.BlockSpec((1, _BM, _N), lambda m, l, i: (m, i, 0)),
            pl.BlockSpec((_NMETA, _NFEAT, _NHID), lambda m, l, i: (0, 0, 0)),
            pl.BlockSpec((_NMETA, 1, _NHID), lambda m, l, i: (0, 0, 0)),
            pl.BlockSpec((_NMETA, _NHID, _DIM_MP), lambda m, l, i: (0, 0, 0)),
            pl.BlockSpec((_NMETA, 1, _DIM_MP), lambda m, l, i: (0, 0, 0)),
            pl.BlockSpec((_DIM_MP, 1), lambda m, l, i: (0, 0)),
            pl.BlockSpec((_DIM_MP, _NCLASS), lambda m, l, i: (0, 0)),
            pl.BlockSpec((1, _NCLASS), lambda m, l, i: (0, 0)),
        ],
        out_specs=pl.BlockSpec((_BM, _NCLASS), lambda m, l, i: (i, 0)),
        out_shape=jax.ShapeDtypeStruct((_N, _NCLASS), jnp.float32),
        compiler_params=pltpu.CompilerParams(
            vmem_limit_bytes=62 * 1024 * 1024,
        ),
        scratch_shapes=[
            pltpu.VMEM((_N, _NHID), jnp.bfloat16),
            pltpu.VMEM((_N, _NHID), jnp.bfloat16),
            pltpu.VMEM((_N, _DIM_MP), jnp.bfloat16),
            pltpu.VMEM((_N, _DIM_MP), jnp.float32),
            pltpu.VMEM((_N, _DIM_MP), jnp.float32),
        ],
    )(input, adjs, W1, b1r, W2, b2r, a_r, Wlin, blin_r)
